# baseline (speedup 1.0000x reference)
import functools

import jax
import jax.numpy as jnp
from jax import lax
from jax.experimental import pallas as pl
from jax.experimental.pallas import tpu as pltpu
from jax.experimental.pallas import tpu_sc as plsc

N = 10000
E = 320000
HID = 64
IN = 128

NC = 2
NS = 16
NW = NC * NS
CHUNK = 8
ROWS_PER_W = 80
ROWS = NW * ROWS_PER_W
EPAD = ROWS * 128
NPAD = 10112
RPT = NPAD // NS

T = 4096
L1 = 5120
L2 = 7168
L1C = L1 // (CHUNK * 128)
L2C = L2 // (CHUNK * 128)
UPT = T // NS

_MESH = plsc.VectorSubcoreMesh(core_axis_name="c", subcore_axis_name="s")


@functools.partial(
    pl.kernel,
    out_type=[
        jax.ShapeDtypeStruct((NW, L1), jnp.int32),
        jax.ShapeDtypeStruct((NW, L1), jnp.int32),
        jax.ShapeDtypeStruct((NW, L2), jnp.int32),
        jax.ShapeDtypeStruct((NW, L2), jnp.int32),
        jax.ShapeDtypeStruct((NC, NPAD, 8), jnp.float32),
    ],
    mesh=_MESH,
    scratch_types=[
        pltpu.VMEM_SHARED((NPAD, 8), jnp.float32),
        pltpu.VMEM((CHUNK, 128), jnp.int32),
        pltpu.VMEM((CHUNK, 128), jnp.int32),
        pltpu.VMEM((128, 8), jnp.float32),
        pltpu.VMEM((L1 + 16,), jnp.int32),
        pltpu.VMEM((L1 + 16,), jnp.int32),
        pltpu.VMEM((L2 + 16,), jnp.int32),
        pltpu.VMEM((L2 + 16,), jnp.int32),
    ],
    compiler_params=pltpu.CompilerParams(use_tc_tiling_on_sc=False,
                                         needs_layout_passes=False),
)
def _sc_partition(src_hbm, dst_hbm, ones_hbm, zero_hbm,
                  losrc_hbm, lodst_hbm, hisrc_hbm, hidst_hbm, deg_hbm,
                  acc, sbuf, dbuf, obuf, ls, ld, hs, hd):
    c = lax.axis_index("c")
    s = lax.axis_index("s")
    w = c * NS + s
    pltpu.sync_copy(zero_hbm, acc.at[pl.ds(s * RPT, RPT)])
    pltpu.sync_copy(ones_hbm, obuf)
    plsc.subcore_barrier()
    row0 = w * ROWS_PER_W

    iota = lax.iota(jnp.int32, 16)
    rank = iota + 1

    def body(i, carry):
        off_lo, off_hi = carry
        base = row0 + i * CHUNK
        pltpu.sync_copy(src_hbm.at[pl.ds(base, CHUNK)], sbuf)
        pltpu.sync_copy(dst_hbm.at[pl.ds(base, CHUNK)], dbuf)
        for j in range(CHUNK):
            pltpu.sync_copy(obuf, acc.at[dbuf.at[j]], add=True)
        for r in range(CHUNK):
            for g in range(8):
                sv = sbuf[r, pl.ds(g * 16, 16)]
                dv = dbuf[r, pl.ds(g * 16, 16)]
                m = sv < T
                cum = plsc.cumsum(m.astype(jnp.int32))
                n = plsc.all_reduce_population_count(m)
                pos_lo = jnp.where(m, off_lo + cum - 1, L1 + 15)
                plsc.store_scatter(ls, [pos_lo], sv)
                plsc.store_scatter(ld, [pos_lo], dv)
                pos_hi = jnp.where(m, L2 + 15, off_hi + (rank - cum) - 1)
                plsc.store_scatter(hs, [pos_hi], sv)
                plsc.store_scatter(hd, [pos_hi], dv)
                off_lo = jnp.minimum(off_lo + n, L1)
                off_hi = jnp.minimum(off_hi + (16 - n), L2)
        return off_lo, off_hi

    zoff = jnp.zeros((16,), jnp.int32)
    off_lo, off_hi = lax.fori_loop(0, ROWS_PER_W // CHUNK, body, (zoff, zoff))

    zpad = jnp.zeros((16,), jnp.int32)
    tpad = jnp.full((16,), N, jnp.int32)

    def padlo(i, off):
        pos = jnp.minimum(off + i * 16 + iota, L1 + 15)
        plsc.store_scatter(ls, [pos], zpad)
        plsc.store_scatter(ld, [pos], tpad)
        return off

    lax.fori_loop(0, L1 // 16 + 1, padlo, off_lo)

    def padhi(i, off):
        pos = jnp.minimum(off + i * 16 + iota, L2 + 15)
        plsc.store_scatter(hs, [pos], zpad)
        plsc.store_scatter(hd, [pos], tpad)
        return off

    lax.fori_loop(0, L2 // 16 + 1, padhi, off_hi)

    pltpu.sync_copy(ls.at[pl.ds(0, L1)], losrc_hbm.at[w])
    pltpu.sync_copy(ld.at[pl.ds(0, L1)], lodst_hbm.at[w])
    pltpu.sync_copy(hs.at[pl.ds(0, L2)], hisrc_hbm.at[w])
    pltpu.sync_copy(hd.at[pl.ds(0, L2)], hidst_hbm.at[w])
    plsc.subcore_barrier()
    pltpu.sync_copy(acc.at[pl.ds(s * RPT, RPT)],
                    deg_hbm.at[c, pl.ds(s * RPT, RPT)])


@functools.partial(
    pl.kernel,
    out_type=jax.ShapeDtypeStruct((NC, NPAD, HID), jnp.float32),
    mesh=_MESH,
    scratch_types=[
        pltpu.VMEM_SHARED((NPAD, HID), jnp.float32),
        pltpu.VMEM_SHARED((T, HID), jnp.float32),
        pltpu.VMEM((CHUNK, 128), jnp.int32),
        pltpu.VMEM((CHUNK, 128), jnp.int32),
        pltpu.VMEM((CHUNK, 128, HID), jnp.float32),
        pltpu.SemaphoreType.DMA,
    ],
    compiler_params=pltpu.CompilerParams(use_tc_tiling_on_sc=False),
)
def _sc_propagate(u_hbm, losrc_hbm, lodst_hbm, hisrc_hbm, hidst_hbm, zero_hbm,
                  out_hbm, acc, u_spm, sbuf, dbuf, gbuf, sem):
    c = lax.axis_index("c")
    s = lax.axis_index("s")
    pltpu.sync_copy(zero_hbm, acc.at[pl.ds(s * RPT, RPT)])

    @pl.when(c == 1)
    def _():
        pltpu.sync_copy(u_hbm.at[pl.ds(s * UPT, UPT)],
                        u_spm.at[pl.ds(s * UPT, UPT)])

    plsc.subcore_barrier()

    def run(src_lists, dst_lists, n_chunks, w2, table):
        def body(i, carry):
            pltpu.sync_copy(src_lists.at[w2, i], sbuf)
            pltpu.sync_copy(dst_lists.at[w2, i], dbuf)
            copies = [
                pltpu.async_copy(table.at[sbuf.at[j]], gbuf.at[j], sem)
                for j in range(CHUNK)
            ]
            for cp in copies:
                cp.wait()
            for j in range(CHUNK):
                pltpu.sync_copy(gbuf.at[j], acc.at[dbuf.at[j]], add=True)
            return carry

        lax.fori_loop(0, n_chunks, body, 0)

    @pl.when(c == 1)
    def _():
        for k in range(2):
            run(losrc_hbm, lodst_hbm, L1C, 2 * s + k, u_spm)

    @pl.when(c == 0)
    def _():
        for k in range(2):
            run(hisrc_hbm, hidst_hbm, L2C, 2 * s + k, u_hbm)

    plsc.subcore_barrier()
    pltpu.sync_copy(acc.at[pl.ds(s * RPT, RPT)],
                    out_hbm.at[c, pl.ds(s * RPT, RPT)])


_BM = 1000


def _row_spec(d):
    return pl.BlockSpec((_BM, d), lambda i: (i, 0))


def _full_spec(r, d):
    return pl.BlockSpec((r, d), lambda i: (0, 0))


def _tc_call(body, in_specs, out_dim, n_out=1):
    if n_out == 1:
        out_shape = jax.ShapeDtypeStruct((N, out_dim), jnp.float32)
        out_specs = _row_spec(out_dim)
    else:
        out_shape = [jax.ShapeDtypeStruct((N, out_dim), jnp.float32)] * n_out
        out_specs = [_row_spec(out_dim)] * n_out
    return pl.pallas_call(
        body,
        grid=(N // _BM,),
        in_specs=in_specs,
        out_specs=out_specs,
        out_shape=out_shape,
    )


def _k_xw1(x_ref, w_ref, o_ref):
    o_ref[...] = jnp.dot(x_ref[...], w_ref[...],
                         preferred_element_type=jnp.float32)


def _k_dinv_u1(p0_ref, p1_ref, t1_ref, dinv_ref, u1_ref):
    deg = p0_ref[:, 0:1] + p1_ref[:, 0:1] + 1.0
    dinv = lax.rsqrt(jnp.broadcast_to(deg, (_BM, HID)))
    dinv_ref[...] = dinv
    u1_ref[...] = t1_ref[...] * dinv


def _k_mid(s0_ref, s1_ref, u_ref, dinv_ref, b_ref, w_ref, o_ref):
    dinv = dinv_ref[...]
    h = dinv * (s0_ref[...] + s1_ref[...] + u_ref[...]) + b_ref[...]
    h = jnp.maximum(h, 0.0)
    o_ref[...] = jnp.dot(h, w_ref[...],
                         preferred_element_type=jnp.float32) * dinv


def _k_emb(s0_ref, s1_ref, u_ref, dinv_ref, b_ref, o_ref):
    dinv = dinv_ref[...]
    emb = dinv * (s0_ref[...] + s1_ref[...] + u_ref[...]) + b_ref[...]
    o_ref[...] = emb * dinv


def _k_out(s0_ref, s1_ref, u_ref, dinv_ref, w_ref, b_ref, o_ref):
    ph = dinv_ref[...] * (s0_ref[...] + s1_ref[...] + u_ref[...])
    o_ref[...] = jnp.dot(ph, w_ref[...],
                         preferred_element_type=jnp.float32) + b_ref[...]


def kernel(x, edge_index, W1, b1, W2, b2, W3, b3, W4, b4):
    ei = edge_index.astype(jnp.int32)
    pad = EPAD - E
    srcp = jnp.concatenate([ei[0], jnp.zeros((pad,), jnp.int32)]).reshape(ROWS, 128)
    pad_dst = N + (jnp.arange(pad, dtype=jnp.int32) % (NPAD - N))
    dstp = jnp.concatenate([ei[1], pad_dst]).reshape(ROWS, 128)
    zeros_h = jnp.zeros((RPT, HID), jnp.float32)
    zeros_8 = jnp.zeros((RPT, 8), jnp.float32)
    ones_8 = jnp.ones((128, 8), jnp.float32)

    losrc, lodst, hisrc, hidst, degp = _sc_partition(srcp, dstp, ones_8, zeros_8)
    losrc = losrc.reshape(NW, L1C, CHUNK, 128)
    lodst = lodst.reshape(NW, L1C, CHUNK, 128)
    hisrc = hisrc.reshape(NW, L2C, CHUNK, 128)
    hidst = hidst.reshape(NW, L2C, CHUNK, 128)

    t1 = _tc_call(_k_xw1, [_row_spec(IN), _full_spec(IN, HID)], HID)(x, W1)
    dinv, u1 = _tc_call(
        _k_dinv_u1, [_row_spec(8), _row_spec(8), _row_spec(HID)], HID, n_out=2,
    )(degp[0], degp[1], t1)

    sp = _sc_propagate(u1, losrc, lodst, hisrc, hidst, zeros_h)
    u2 = _tc_call(
        _k_mid,
        [_row_spec(HID)] * 4 + [_full_spec(1, HID), _full_spec(HID, HID)],
        HID,
    )(sp[0], sp[1], u1, dinv, b1.reshape(1, HID), W2)

    sp = _sc_propagate(u2, losrc, lodst, hisrc, hidst, zeros_h)
    u3 = _tc_call(
        _k_mid,
        [_row_spec(HID)] * 4 + [_full_spec(1, HID), _full_spec(HID, HID)],
        HID,
    )(sp[0], sp[1], u2, dinv, b2.reshape(1, HID), W3)

    sp = _sc_propagate(u3, losrc, lodst, hisrc, hidst, zeros_h)
    u4 = _tc_call(
        _k_emb, [_row_spec(HID)] * 4 + [_full_spec(1, HID)], HID,
    )(sp[0], sp[1], u3, dinv, b3.reshape(1, HID))

    sp = _sc_propagate(u4, losrc, lodst, hisrc, hidst, zeros_h)
    x_ = _tc_call(
        _k_out,
        [_row_spec(HID)] * 4 + [_full_spec(HID, IN), _full_spec(1, IN)],
        IN,
    )(sp[0], sp[1], u4, dinv, W4, b4.reshape(1, IN))
    return x_


# speedup vs baseline: 5.3001x; 5.3001x over previous
import functools

import jax
import jax.numpy as jnp
from jax import lax
from jax.experimental import pallas as pl
from jax.experimental.pallas import tpu as pltpu
from jax.experimental.pallas import tpu_sc as plsc

N = 10000
E = 320000
HID = 64
IN = 128

NC = 2
NS = 16
NW = NC * NS
CHUNK = 8
ROWS_PER_W = 80
ROWS = NW * ROWS_PER_W
EPAD = ROWS * 128
NPAD = 10112
RPT = NPAD // NS

T = 4096
L1 = 5120
L2 = 7168
L1C = L1 // (CHUNK * 128)
L2C = L2 // (CHUNK * 128)
UPT = T // NS

_MESH = plsc.VectorSubcoreMesh(core_axis_name="c", subcore_axis_name="s")


@functools.partial(
    pl.kernel,
    out_type=[
        jax.ShapeDtypeStruct((NW, L1), jnp.int32),
        jax.ShapeDtypeStruct((NW, L1), jnp.int32),
        jax.ShapeDtypeStruct((NW, L2), jnp.int32),
        jax.ShapeDtypeStruct((NW, L2), jnp.int32),
        jax.ShapeDtypeStruct((NC, NPAD, 8), jnp.float32),
    ],
    mesh=_MESH,
    scratch_types=[
        pltpu.VMEM_SHARED((NPAD, 8), jnp.float32),
        pltpu.VMEM((CHUNK, 128), jnp.int32),
        pltpu.VMEM((CHUNK, 128), jnp.int32),
        pltpu.VMEM((128, 8), jnp.float32),
        pltpu.VMEM((L1 + 16,), jnp.int32),
        pltpu.VMEM((L1 + 16,), jnp.int32),
        pltpu.VMEM((L2 + 16,), jnp.int32),
        pltpu.VMEM((L2 + 16,), jnp.int32),
    ],
    compiler_params=pltpu.CompilerParams(use_tc_tiling_on_sc=False,
                                         needs_layout_passes=False),
)
def _sc_partition(src_hbm, dst_hbm, ones_hbm, zero_hbm,
                  losrc_hbm, lodst_hbm, hisrc_hbm, hidst_hbm, deg_hbm,
                  acc, sbuf, dbuf, obuf, ls, ld, hs, hd):
    c = lax.axis_index("c")
    s = lax.axis_index("s")
    w = c * NS + s
    pltpu.sync_copy(zero_hbm, acc.at[pl.ds(s * RPT, RPT)])
    pltpu.sync_copy(ones_hbm, obuf)
    plsc.subcore_barrier()
    row0 = w * ROWS_PER_W

    iota = lax.iota(jnp.int32, 16)
    rank = iota + 1

    def body(i, carry):
        off_lo, off_hi = carry
        base = row0 + i * CHUNK
        pltpu.sync_copy(src_hbm.at[pl.ds(base, CHUNK)], sbuf)
        pltpu.sync_copy(dst_hbm.at[pl.ds(base, CHUNK)], dbuf)
        for j in range(CHUNK):
            pltpu.sync_copy(obuf, acc.at[dbuf.at[j]], add=True)
        for r in range(CHUNK):
            for g in range(8):
                sv = sbuf[r, pl.ds(g * 16, 16)]
                dv = dbuf[r, pl.ds(g * 16, 16)]
                m = sv < T
                cum = plsc.cumsum(m.astype(jnp.int32))
                n = plsc.all_reduce_population_count(m)
                pos_lo = jnp.where(m, off_lo + cum - 1, L1 + 15)
                plsc.store_scatter(ls, [pos_lo], sv)
                plsc.store_scatter(ld, [pos_lo], dv)
                pos_hi = jnp.where(m, L2 + 15, off_hi + (rank - cum) - 1)
                plsc.store_scatter(hs, [pos_hi], sv)
                plsc.store_scatter(hd, [pos_hi], dv)
                off_lo = jnp.minimum(off_lo + n, L1)
                off_hi = jnp.minimum(off_hi + (16 - n), L2)
        return off_lo, off_hi

    zoff = jnp.zeros((16,), jnp.int32)
    off_lo, off_hi = lax.fori_loop(0, ROWS_PER_W // CHUNK, body, (zoff, zoff))

    def padlo(i, off):
        pos = jnp.minimum(off + i * 16 + iota, L1 + 15)
        sp_ = (i * 16 + iota) % T
        dp_ = N + ((i * 16 + iota) % (NPAD - N))
        plsc.store_scatter(ls, [pos], sp_)
        plsc.store_scatter(ld, [pos], dp_)
        return off

    lax.fori_loop(0, L1 // 16 + 1, padlo, off_lo)

    def padhi(i, off):
        pos = jnp.minimum(off + i * 16 + iota, L2 + 15)
        sp_ = T + ((i * 16 + iota) % (N - T))
        dp_ = N + ((i * 16 + iota) % (NPAD - N))
        plsc.store_scatter(hs, [pos], sp_)
        plsc.store_scatter(hd, [pos], dp_)
        return off

    lax.fori_loop(0, L2 // 16 + 1, padhi, off_hi)

    pltpu.sync_copy(ls.at[pl.ds(0, L1)], losrc_hbm.at[w])
    pltpu.sync_copy(ld.at[pl.ds(0, L1)], lodst_hbm.at[w])
    pltpu.sync_copy(hs.at[pl.ds(0, L2)], hisrc_hbm.at[w])
    pltpu.sync_copy(hd.at[pl.ds(0, L2)], hidst_hbm.at[w])
    plsc.subcore_barrier()
    pltpu.sync_copy(acc.at[pl.ds(s * RPT, RPT)],
                    deg_hbm.at[c, pl.ds(s * RPT, RPT)])


@functools.partial(
    pl.kernel,
    out_type=jax.ShapeDtypeStruct((NC, NPAD, HID), jnp.float32),
    mesh=_MESH,
    scratch_types=[
        pltpu.VMEM_SHARED((NPAD, HID), jnp.float32),
        pltpu.VMEM_SHARED((T, HID), jnp.float32),
        pltpu.VMEM((CHUNK, 128), jnp.int32),
        pltpu.VMEM((CHUNK, 128), jnp.int32),
        pltpu.VMEM((CHUNK, 128, HID), jnp.float32),
        pltpu.SemaphoreType.DMA,
    ],
    compiler_params=pltpu.CompilerParams(use_tc_tiling_on_sc=False),
)
def _sc_propagate(u_hbm, losrc_hbm, lodst_hbm, hisrc_hbm, hidst_hbm, zero_hbm,
                  out_hbm, acc, u_spm, sbuf, dbuf, gbuf, sem):
    c = lax.axis_index("c")
    s = lax.axis_index("s")
    pltpu.sync_copy(zero_hbm, acc.at[pl.ds(s * RPT, RPT)])

    @pl.when(c == 1)
    def _():
        pltpu.sync_copy(u_hbm.at[pl.ds(s * UPT, UPT)],
                        u_spm.at[pl.ds(s * UPT, UPT)])

    plsc.subcore_barrier()

    def run(src_lists, dst_lists, n_chunks, w2, table):
        def body(i, carry):
            pltpu.sync_copy(src_lists.at[w2, i], sbuf)
            pltpu.sync_copy(dst_lists.at[w2, i], dbuf)
            copies = [
                pltpu.async_copy(table.at[sbuf.at[j]], gbuf.at[j], sem)
                for j in range(CHUNK)
            ]
            for cp in copies:
                cp.wait()
            for j in range(CHUNK):
                pltpu.sync_copy(gbuf.at[j], acc.at[dbuf.at[j]], add=True)
            return carry

        lax.fori_loop(0, n_chunks, body, 0)

    @pl.when(c == 1)
    def _():
        for k in range(2):
            run(losrc_hbm, lodst_hbm, L1C, 2 * s + k, u_spm)

    @pl.when(c == 0)
    def _():
        for k in range(2):
            run(hisrc_hbm, hidst_hbm, L2C, 2 * s + k, u_hbm)

    plsc.subcore_barrier()
    pltpu.sync_copy(acc.at[pl.ds(s * RPT, RPT)],
                    out_hbm.at[c, pl.ds(s * RPT, RPT)])


_BM = 1000


def _row_spec(d):
    return pl.BlockSpec((_BM, d), lambda i: (i, 0))


def _full_spec(r, d):
    return pl.BlockSpec((r, d), lambda i: (0, 0))


def _tc_call(body, in_specs, out_dim, n_out=1):
    if n_out == 1:
        out_shape = jax.ShapeDtypeStruct((N, out_dim), jnp.float32)
        out_specs = _row_spec(out_dim)
    else:
        out_shape = [jax.ShapeDtypeStruct((N, out_dim), jnp.float32)] * n_out
        out_specs = [_row_spec(out_dim)] * n_out
    return pl.pallas_call(
        body,
        grid=(N // _BM,),
        in_specs=in_specs,
        out_specs=out_specs,
        out_shape=out_shape,
    )


def _k_xw1(x_ref, w_ref, o_ref):
    o_ref[...] = jnp.dot(x_ref[...], w_ref[...],
                         preferred_element_type=jnp.float32)


def _k_dinv_u1(p0_ref, p1_ref, t1_ref, dinv_ref, u1_ref):
    deg = p0_ref[:, 0:1] + p1_ref[:, 0:1] + 1.0
    dinv = lax.rsqrt(jnp.broadcast_to(deg, (_BM, HID)))
    dinv_ref[...] = dinv
    u1_ref[...] = t1_ref[...] * dinv


def _k_mid(s0_ref, s1_ref, u_ref, dinv_ref, b_ref, w_ref, o_ref):
    dinv = dinv_ref[...]
    h = dinv * (s0_ref[...] + s1_ref[...] + u_ref[...]) + b_ref[...]
    h = jnp.maximum(h, 0.0)
    o_ref[...] = jnp.dot(h, w_ref[...],
                         preferred_element_type=jnp.float32) * dinv


def _k_emb(s0_ref, s1_ref, u_ref, dinv_ref, b_ref, o_ref):
    dinv = dinv_ref[...]
    emb = dinv * (s0_ref[...] + s1_ref[...] + u_ref[...]) + b_ref[...]
    o_ref[...] = emb * dinv


def _k_out(s0_ref, s1_ref, u_ref, dinv_ref, w_ref, b_ref, o_ref):
    ph = dinv_ref[...] * (s0_ref[...] + s1_ref[...] + u_ref[...])
    o_ref[...] = jnp.dot(ph, w_ref[...],
                         preferred_element_type=jnp.float32) + b_ref[...]


def kernel(x, edge_index, W1, b1, W2, b2, W3, b3, W4, b4):
    ei = edge_index.astype(jnp.int32)
    pad = EPAD - E
    pad_src = jnp.arange(pad, dtype=jnp.int32) % T
    srcp = jnp.concatenate([ei[0], pad_src]).reshape(ROWS, 128)
    pad_dst = N + (jnp.arange(pad, dtype=jnp.int32) % (NPAD - N))
    dstp = jnp.concatenate([ei[1], pad_dst]).reshape(ROWS, 128)
    zeros_h = jnp.zeros((RPT, HID), jnp.float32)
    zeros_8 = jnp.zeros((RPT, 8), jnp.float32)
    ones_8 = jnp.ones((128, 8), jnp.float32)

    losrc, lodst, hisrc, hidst, degp = _sc_partition(srcp, dstp, ones_8, zeros_8)
    losrc = losrc.reshape(NW, L1C, CHUNK, 128)
    lodst = lodst.reshape(NW, L1C, CHUNK, 128)
    hisrc = hisrc.reshape(NW, L2C, CHUNK, 128)
    hidst = hidst.reshape(NW, L2C, CHUNK, 128)

    t1 = _tc_call(_k_xw1, [_row_spec(IN), _full_spec(IN, HID)], HID)(x, W1)
    dinv, u1 = _tc_call(
        _k_dinv_u1, [_row_spec(8), _row_spec(8), _row_spec(HID)], HID, n_out=2,
    )(degp[0], degp[1], t1)

    sp = _sc_propagate(u1, losrc, lodst, hisrc, hidst, zeros_h)
    u2 = _tc_call(
        _k_mid,
        [_row_spec(HID)] * 4 + [_full_spec(1, HID), _full_spec(HID, HID)],
        HID,
    )(sp[0], sp[1], u1, dinv, b1.reshape(1, HID), W2)

    sp = _sc_propagate(u2, losrc, lodst, hisrc, hidst, zeros_h)
    u3 = _tc_call(
        _k_mid,
        [_row_spec(HID)] * 4 + [_full_spec(1, HID), _full_spec(HID, HID)],
        HID,
    )(sp[0], sp[1], u2, dinv, b2.reshape(1, HID), W3)

    sp = _sc_propagate(u3, losrc, lodst, hisrc, hidst, zeros_h)
    u4 = _tc_call(
        _k_emb, [_row_spec(HID)] * 4 + [_full_spec(1, HID)], HID,
    )(sp[0], sp[1], u3, dinv, b3.reshape(1, HID))

    sp = _sc_propagate(u4, losrc, lodst, hisrc, hidst, zeros_h)
    x_ = _tc_call(
        _k_out,
        [_row_spec(HID)] * 4 + [_full_spec(HID, IN), _full_spec(1, IN)],
        IN,
    )(sp[0], sp[1], u4, dinv, W4, b4.reshape(1, IN))
    return x_


# speedup vs baseline: 5.7755x; 1.0897x over previous
import functools

import jax
import jax.numpy as jnp
from jax import lax
from jax.experimental import pallas as pl
from jax.experimental.pallas import tpu as pltpu
from jax.experimental.pallas import tpu_sc as plsc

N = 10000
E = 320000
HID = 64
IN = 128

NC = 2
NS = 16
NW = NC * NS
CHUNK = 8
ROWS_PER_W = 80
ROWS = NW * ROWS_PER_W
EPAD = ROWS * 128
NPAD = 10112
RPT = NPAD // NS

T = 4096
L1 = 5120
L2 = 7168
PCH = 4
L1C = L1 // (PCH * 128)
L2C = L2 // (PCH * 128)
UPT = T // NS

_MESH = plsc.VectorSubcoreMesh(core_axis_name="c", subcore_axis_name="s")


@functools.partial(
    pl.kernel,
    out_type=[
        jax.ShapeDtypeStruct((NW, L1), jnp.int32),
        jax.ShapeDtypeStruct((NW, L1), jnp.int32),
        jax.ShapeDtypeStruct((NW, L2), jnp.int32),
        jax.ShapeDtypeStruct((NW, L2), jnp.int32),
        jax.ShapeDtypeStruct((NC, NPAD, 8), jnp.float32),
    ],
    mesh=_MESH,
    scratch_types=[
        pltpu.VMEM_SHARED((NPAD, 8), jnp.float32),
        pltpu.VMEM((CHUNK, 128), jnp.int32),
        pltpu.VMEM((CHUNK, 128), jnp.int32),
        pltpu.VMEM((128, 8), jnp.float32),
        pltpu.VMEM((L1 + 16,), jnp.int32),
        pltpu.VMEM((L1 + 16,), jnp.int32),
        pltpu.VMEM((L2 + 16,), jnp.int32),
        pltpu.VMEM((L2 + 16,), jnp.int32),
    ],
    compiler_params=pltpu.CompilerParams(use_tc_tiling_on_sc=False,
                                         needs_layout_passes=False),
)
def _sc_partition(src_hbm, dst_hbm, ones_hbm, zero_hbm,
                  losrc_hbm, lodst_hbm, hisrc_hbm, hidst_hbm, deg_hbm,
                  acc, sbuf, dbuf, obuf, ls, ld, hs, hd):
    c = lax.axis_index("c")
    s = lax.axis_index("s")
    w = c * NS + s
    pltpu.sync_copy(zero_hbm, acc.at[pl.ds(s * RPT, RPT)])
    pltpu.sync_copy(ones_hbm, obuf)
    plsc.subcore_barrier()
    row0 = w * ROWS_PER_W

    iota = lax.iota(jnp.int32, 16)
    rank = iota + 1

    def body(i, carry):
        off_lo, off_hi = carry
        base = row0 + i * CHUNK
        pltpu.sync_copy(src_hbm.at[pl.ds(base, CHUNK)], sbuf)
        pltpu.sync_copy(dst_hbm.at[pl.ds(base, CHUNK)], dbuf)
        for j in range(CHUNK):
            pltpu.sync_copy(obuf, acc.at[dbuf.at[j]], add=True)
        for r in range(CHUNK):
            for g in range(8):
                sv = sbuf[r, pl.ds(g * 16, 16)]
                dv = dbuf[r, pl.ds(g * 16, 16)]
                m = sv < T
                cum = plsc.cumsum(m.astype(jnp.int32))
                n = plsc.all_reduce_population_count(m)
                pos_lo = jnp.where(m, off_lo + cum - 1, L1 + 15)
                plsc.store_scatter(ls, [pos_lo], sv)
                plsc.store_scatter(ld, [pos_lo], dv)
                pos_hi = jnp.where(m, L2 + 15, off_hi + (rank - cum) - 1)
                plsc.store_scatter(hs, [pos_hi], sv)
                plsc.store_scatter(hd, [pos_hi], dv)
                off_lo = jnp.minimum(off_lo + n, L1)
                off_hi = jnp.minimum(off_hi + (16 - n), L2)
        return off_lo, off_hi

    zoff = jnp.zeros((16,), jnp.int32)
    off_lo, off_hi = lax.fori_loop(0, ROWS_PER_W // CHUNK, body, (zoff, zoff))

    def padlo(i, off):
        pos = jnp.minimum(off + i * 16 + iota, L1 + 15)
        sp_ = (i * 16 + iota) % T
        dp_ = N + ((i * 16 + iota) % (NPAD - N))
        plsc.store_scatter(ls, [pos], sp_)
        plsc.store_scatter(ld, [pos], dp_)
        return off

    lax.fori_loop(0, L1 // 16 + 1, padlo, off_lo)

    def padhi(i, off):
        pos = jnp.minimum(off + i * 16 + iota, L2 + 15)
        sp_ = T + ((i * 16 + iota) % (N - T))
        dp_ = N + ((i * 16 + iota) % (NPAD - N))
        plsc.store_scatter(hs, [pos], sp_)
        plsc.store_scatter(hd, [pos], dp_)
        return off

    lax.fori_loop(0, L2 // 16 + 1, padhi, off_hi)

    pltpu.sync_copy(ls.at[pl.ds(0, L1)], losrc_hbm.at[w])
    pltpu.sync_copy(ld.at[pl.ds(0, L1)], lodst_hbm.at[w])
    pltpu.sync_copy(hs.at[pl.ds(0, L2)], hisrc_hbm.at[w])
    pltpu.sync_copy(hd.at[pl.ds(0, L2)], hidst_hbm.at[w])
    plsc.subcore_barrier()
    pltpu.sync_copy(acc.at[pl.ds(s * RPT, RPT)],
                    deg_hbm.at[c, pl.ds(s * RPT, RPT)])


@functools.partial(
    pl.kernel,
    out_type=jax.ShapeDtypeStruct((NC, NPAD, HID), jnp.float32),
    mesh=_MESH,
    scratch_types=[
        pltpu.VMEM_SHARED((NPAD, HID), jnp.float32),
        pltpu.VMEM_SHARED((T, HID), jnp.float32),
        pltpu.VMEM((PCH, 128), jnp.int32),
        pltpu.VMEM((2, PCH, 128), jnp.int32),
        pltpu.VMEM((2, PCH, 128, HID), jnp.float32),
        pltpu.SemaphoreType.DMA,
        pltpu.SemaphoreType.DMA,
    ],
    compiler_params=pltpu.CompilerParams(use_tc_tiling_on_sc=False),
)
def _sc_propagate(u_hbm, losrc_hbm, lodst_hbm, hisrc_hbm, hidst_hbm, zero_hbm,
                  out_hbm, acc, u_spm, sbuf, dbuf, gbuf, gsem, ssem):
    c = lax.axis_index("c")
    s = lax.axis_index("s")
    pltpu.sync_copy(zero_hbm, acc.at[pl.ds(s * RPT, RPT)])

    @pl.when(c == 1)
    def _():
        pltpu.sync_copy(u_hbm.at[pl.ds(s * UPT, UPT)],
                        u_spm.at[pl.ds(s * UPT, UPT)])

    plsc.subcore_barrier()

    def run(src_lists, dst_lists, n_chunks, table):
        pend = [None, None]
        seq = [(2 * s + k, i) for k in range(2) for i in range(n_chunks)]
        for t, (w2, i) in enumerate(seq):
            p = t % 2
            if pend[p] is not None:
                for cp in pend[p]:
                    cp.wait()
            pltpu.sync_copy(src_lists.at[w2, i], sbuf)
            pltpu.sync_copy(dst_lists.at[w2, i], dbuf.at[p])
            gs = [
                pltpu.async_copy(table.at[sbuf.at[j]], gbuf.at[p, j], gsem)
                for j in range(PCH)
            ]
            for cp in gs:
                cp.wait()
            pend[p] = [
                pltpu.async_copy(gbuf.at[p, j], acc.at[dbuf.at[p, j]],
                                 ssem, add=True)
                for j in range(PCH)
            ]
        for ss in pend:
            if ss is not None:
                for cp in ss:
                    cp.wait()

    @pl.when(c == 1)
    def _():
        run(losrc_hbm, lodst_hbm, L1C, u_spm)

    @pl.when(c == 0)
    def _():
        run(hisrc_hbm, hidst_hbm, L2C, u_hbm)

    plsc.subcore_barrier()
    pltpu.sync_copy(acc.at[pl.ds(s * RPT, RPT)],
                    out_hbm.at[c, pl.ds(s * RPT, RPT)])


_BM = 1000


def _row_spec(d):
    return pl.BlockSpec((_BM, d), lambda i: (i, 0))


def _full_spec(r, d):
    return pl.BlockSpec((r, d), lambda i: (0, 0))


def _tc_call(body, in_specs, out_dim, n_out=1):
    if n_out == 1:
        out_shape = jax.ShapeDtypeStruct((N, out_dim), jnp.float32)
        out_specs = _row_spec(out_dim)
    else:
        out_shape = [jax.ShapeDtypeStruct((N, out_dim), jnp.float32)] * n_out
        out_specs = [_row_spec(out_dim)] * n_out
    return pl.pallas_call(
        body,
        grid=(N // _BM,),
        in_specs=in_specs,
        out_specs=out_specs,
        out_shape=out_shape,
    )


def _k_xw1(x_ref, w_ref, o_ref):
    o_ref[...] = jnp.dot(x_ref[...], w_ref[...],
                         preferred_element_type=jnp.float32)


def _k_dinv_u1(p0_ref, p1_ref, t1_ref, dinv_ref, u1_ref):
    deg = p0_ref[:, 0:1] + p1_ref[:, 0:1] + 1.0
    dinv = lax.rsqrt(jnp.broadcast_to(deg, (_BM, HID)))
    dinv_ref[...] = dinv
    u1_ref[...] = t1_ref[...] * dinv


def _k_mid(s0_ref, s1_ref, u_ref, dinv_ref, b_ref, w_ref, o_ref):
    dinv = dinv_ref[...]
    h = dinv * (s0_ref[...] + s1_ref[...] + u_ref[...]) + b_ref[...]
    h = jnp.maximum(h, 0.0)
    o_ref[...] = jnp.dot(h, w_ref[...],
                         preferred_element_type=jnp.float32) * dinv


def _k_emb(s0_ref, s1_ref, u_ref, dinv_ref, b_ref, o_ref):
    dinv = dinv_ref[...]
    emb = dinv * (s0_ref[...] + s1_ref[...] + u_ref[...]) + b_ref[...]
    o_ref[...] = emb * dinv


def _k_out(s0_ref, s1_ref, u_ref, dinv_ref, w_ref, b_ref, o_ref):
    ph = dinv_ref[...] * (s0_ref[...] + s1_ref[...] + u_ref[...])
    o_ref[...] = jnp.dot(ph, w_ref[...],
                         preferred_element_type=jnp.float32) + b_ref[...]


def kernel(x, edge_index, W1, b1, W2, b2, W3, b3, W4, b4):
    ei = edge_index.astype(jnp.int32)
    pad = EPAD - E
    pad_src = jnp.arange(pad, dtype=jnp.int32) % T
    srcp = jnp.concatenate([ei[0], pad_src]).reshape(ROWS, 128)
    pad_dst = N + (jnp.arange(pad, dtype=jnp.int32) % (NPAD - N))
    dstp = jnp.concatenate([ei[1], pad_dst]).reshape(ROWS, 128)
    zeros_h = jnp.zeros((RPT, HID), jnp.float32)
    zeros_8 = jnp.zeros((RPT, 8), jnp.float32)
    ones_8 = jnp.ones((128, 8), jnp.float32)

    losrc, lodst, hisrc, hidst, degp = _sc_partition(srcp, dstp, ones_8, zeros_8)
    losrc = losrc.reshape(NW, L1C, PCH, 128)
    lodst = lodst.reshape(NW, L1C, PCH, 128)
    hisrc = hisrc.reshape(NW, L2C, PCH, 128)
    hidst = hidst.reshape(NW, L2C, PCH, 128)

    t1 = _tc_call(_k_xw1, [_row_spec(IN), _full_spec(IN, HID)], HID)(x, W1)
    dinv, u1 = _tc_call(
        _k_dinv_u1, [_row_spec(8), _row_spec(8), _row_spec(HID)], HID, n_out=2,
    )(degp[0], degp[1], t1)

    sp = _sc_propagate(u1, losrc, lodst, hisrc, hidst, zeros_h)
    u2 = _tc_call(
        _k_mid,
        [_row_spec(HID)] * 4 + [_full_spec(1, HID), _full_spec(HID, HID)],
        HID,
    )(sp[0], sp[1], u1, dinv, b1.reshape(1, HID), W2)

    sp = _sc_propagate(u2, losrc, lodst, hisrc, hidst, zeros_h)
    u3 = _tc_call(
        _k_mid,
        [_row_spec(HID)] * 4 + [_full_spec(1, HID), _full_spec(HID, HID)],
        HID,
    )(sp[0], sp[1], u2, dinv, b2.reshape(1, HID), W3)

    sp = _sc_propagate(u3, losrc, lodst, hisrc, hidst, zeros_h)
    u4 = _tc_call(
        _k_emb, [_row_spec(HID)] * 4 + [_full_spec(1, HID)], HID,
    )(sp[0], sp[1], u3, dinv, b3.reshape(1, HID))

    sp = _sc_propagate(u4, losrc, lodst, hisrc, hidst, zeros_h)
    x_ = _tc_call(
        _k_out,
        [_row_spec(HID)] * 4 + [_full_spec(HID, IN), _full_spec(1, IN)],
        IN,
    )(sp[0], sp[1], u4, dinv, W4, b4.reshape(1, IN))
    return x_


# speedup vs baseline: 6.0894x; 1.0543x over previous
import functools

import jax
import jax.numpy as jnp
from jax import lax
from jax.experimental import pallas as pl
from jax.experimental.pallas import tpu as pltpu
from jax.experimental.pallas import tpu_sc as plsc

N = 10000
E = 320000
HID = 64
IN = 128

NC = 2
NS = 16
NW = NC * NS
CHUNK = 8
ROWS_PER_W = 80
ROWS = NW * ROWS_PER_W
EPAD = ROWS * 128
NPAD = 10112
RPT = NPAD // NS

T = 4096
L1 = 5120
L2 = 6656
PCH = 4
L1C = L1 // (PCH * 128)
L2C = L2 // (PCH * 128)
UPT = T // NS

_MESH = plsc.VectorSubcoreMesh(core_axis_name="c", subcore_axis_name="s")


@functools.partial(
    pl.kernel,
    out_type=[
        jax.ShapeDtypeStruct((NW, L1), jnp.int32),
        jax.ShapeDtypeStruct((NW, L1), jnp.int32),
        jax.ShapeDtypeStruct((NW, L2), jnp.int32),
        jax.ShapeDtypeStruct((NW, L2), jnp.int32),
        jax.ShapeDtypeStruct((NC, NPAD, 8), jnp.float32),
    ],
    mesh=_MESH,
    scratch_types=[
        pltpu.VMEM_SHARED((NPAD, 8), jnp.float32),
        pltpu.VMEM((CHUNK, 128), jnp.int32),
        pltpu.VMEM((CHUNK, 128), jnp.int32),
        pltpu.VMEM((128, 8), jnp.float32),
        pltpu.VMEM((L1 + 16,), jnp.int32),
        pltpu.VMEM((L1 + 16,), jnp.int32),
        pltpu.VMEM((L2 + 16,), jnp.int32),
        pltpu.VMEM((L2 + 16,), jnp.int32),
        pltpu.SemaphoreType.DMA,
    ],
    compiler_params=pltpu.CompilerParams(use_tc_tiling_on_sc=False,
                                         needs_layout_passes=False),
)
def _sc_partition(src_hbm, dst_hbm, ones_hbm, zero_hbm,
                  losrc_hbm, lodst_hbm, hisrc_hbm, hidst_hbm, deg_hbm,
                  acc, sbuf, dbuf, obuf, ls, ld, hs, hd, dsem):
    c = lax.axis_index("c")
    s = lax.axis_index("s")
    w = c * NS + s
    pltpu.sync_copy(zero_hbm, acc.at[pl.ds(s * RPT, RPT)])
    pltpu.sync_copy(ones_hbm, obuf)
    plsc.subcore_barrier()
    row0 = w * ROWS_PER_W

    iota = lax.iota(jnp.int32, 16)
    rank = iota + 1

    def body(i, carry):
        off_lo, off_hi = carry
        base = row0 + i * CHUNK
        pltpu.sync_copy(src_hbm.at[pl.ds(base, CHUNK)], sbuf)
        pltpu.sync_copy(dst_hbm.at[pl.ds(base, CHUNK)], dbuf)
        degs = [
            pltpu.async_copy(obuf, acc.at[dbuf.at[j]], dsem, add=True)
            for j in range(CHUNK)
        ]
        for r in range(CHUNK):
            for g in range(8):
                sv = sbuf[r, pl.ds(g * 16, 16)]
                dv = dbuf[r, pl.ds(g * 16, 16)]
                m = sv < T
                cum = plsc.cumsum(m.astype(jnp.int32))
                n = plsc.all_reduce_population_count(m)
                pos_lo = jnp.where(m, off_lo + cum - 1, L1 + 15)
                plsc.store_scatter(ls, [pos_lo], sv)
                plsc.store_scatter(ld, [pos_lo], dv)
                pos_hi = jnp.where(m, L2 + 15, off_hi + (rank - cum) - 1)
                plsc.store_scatter(hs, [pos_hi], sv)
                plsc.store_scatter(hd, [pos_hi], dv)
                off_lo = jnp.minimum(off_lo + n, L1)
                off_hi = jnp.minimum(off_hi + (16 - n), L2)
        for cp in degs:
            cp.wait()
        return off_lo, off_hi

    zoff = jnp.zeros((16,), jnp.int32)
    off_lo, off_hi = lax.fori_loop(0, ROWS_PER_W // CHUNK, body, (zoff, zoff))

    def padlo(i, off):
        pos = jnp.minimum(off + i * 16 + iota, L1 + 15)
        sp_ = (i * 16 + iota) % T
        dp_ = N + ((i * 16 + iota) % (NPAD - N))
        plsc.store_scatter(ls, [pos], sp_)
        plsc.store_scatter(ld, [pos], dp_)
        return off

    lax.fori_loop(0, L1 // 16 + 1, padlo, off_lo)

    def padhi(i, off):
        pos = jnp.minimum(off + i * 16 + iota, L2 + 15)
        sp_ = T + ((i * 16 + iota) % (N - T))
        dp_ = N + ((i * 16 + iota) % (NPAD - N))
        plsc.store_scatter(hs, [pos], sp_)
        plsc.store_scatter(hd, [pos], dp_)
        return off

    lax.fori_loop(0, L2 // 16 + 1, padhi, off_hi)

    pltpu.sync_copy(ls.at[pl.ds(0, L1)], losrc_hbm.at[w])
    pltpu.sync_copy(ld.at[pl.ds(0, L1)], lodst_hbm.at[w])
    pltpu.sync_copy(hs.at[pl.ds(0, L2)], hisrc_hbm.at[w])
    pltpu.sync_copy(hd.at[pl.ds(0, L2)], hidst_hbm.at[w])
    plsc.subcore_barrier()
    pltpu.sync_copy(acc.at[pl.ds(s * RPT, RPT)],
                    deg_hbm.at[c, pl.ds(s * RPT, RPT)])


@functools.partial(
    pl.kernel,
    out_type=jax.ShapeDtypeStruct((NC, NPAD, HID), jnp.float32),
    mesh=_MESH,
    scratch_types=[
        pltpu.VMEM_SHARED((NPAD, HID), jnp.float32),
        pltpu.VMEM_SHARED((T, HID), jnp.float32),
        pltpu.VMEM((PCH, 128), jnp.int32),
        pltpu.VMEM((2, PCH, 128), jnp.int32),
        pltpu.VMEM((2, PCH, 128, HID), jnp.float32),
        pltpu.SemaphoreType.DMA,
        pltpu.SemaphoreType.DMA,
    ],
    compiler_params=pltpu.CompilerParams(use_tc_tiling_on_sc=False),
)
def _sc_propagate(u_hbm, losrc_hbm, lodst_hbm, hisrc_hbm, hidst_hbm, zero_hbm,
                  out_hbm, acc, u_spm, sbuf, dbuf, gbuf, gsem, ssem):
    c = lax.axis_index("c")
    s = lax.axis_index("s")
    pltpu.sync_copy(zero_hbm, acc.at[pl.ds(s * RPT, RPT)])

    @pl.when(c == 1)
    def _():
        pltpu.sync_copy(u_hbm.at[pl.ds(s * UPT, UPT)],
                        u_spm.at[pl.ds(s * UPT, UPT)])

    plsc.subcore_barrier()

    def run(src_lists, dst_lists, n_chunks, table):
        pend = [None, None]
        seq = [(2 * s + k, i) for k in range(2) for i in range(n_chunks)]
        for t, (w2, i) in enumerate(seq):
            p = t % 2
            if pend[p] is not None:
                for cp in pend[p]:
                    cp.wait()
            pltpu.sync_copy(src_lists.at[w2, i], sbuf)
            pltpu.sync_copy(dst_lists.at[w2, i], dbuf.at[p])
            gs = [
                pltpu.async_copy(table.at[sbuf.at[j]], gbuf.at[p, j], gsem)
                for j in range(PCH)
            ]
            for cp in gs:
                cp.wait()
            pend[p] = [
                pltpu.async_copy(gbuf.at[p, j], acc.at[dbuf.at[p, j]],
                                 ssem, add=True)
                for j in range(PCH)
            ]
        for ss in pend:
            if ss is not None:
                for cp in ss:
                    cp.wait()

    @pl.when(c == 1)
    def _():
        run(losrc_hbm, lodst_hbm, L1C, u_spm)

    @pl.when(c == 0)
    def _():
        run(hisrc_hbm, hidst_hbm, L2C, u_hbm)

    plsc.subcore_barrier()
    pltpu.sync_copy(acc.at[pl.ds(s * RPT, RPT)],
                    out_hbm.at[c, pl.ds(s * RPT, RPT)])


_BM = 1000


def _row_spec(d):
    return pl.BlockSpec((_BM, d), lambda i: (i, 0))


def _full_spec(r, d):
    return pl.BlockSpec((r, d), lambda i: (0, 0))


def _tc_call(body, in_specs, out_dim, n_out=1):
    if n_out == 1:
        out_shape = jax.ShapeDtypeStruct((N, out_dim), jnp.float32)
        out_specs = _row_spec(out_dim)
    else:
        out_shape = [jax.ShapeDtypeStruct((N, out_dim), jnp.float32)] * n_out
        out_specs = [_row_spec(out_dim)] * n_out
    return pl.pallas_call(
        body,
        grid=(N // _BM,),
        in_specs=in_specs,
        out_specs=out_specs,
        out_shape=out_shape,
    )


def _k_xw1(x_ref, w_ref, o_ref):
    o_ref[...] = jnp.dot(x_ref[...], w_ref[...],
                         preferred_element_type=jnp.float32)


def _k_dinv_u1(p0_ref, p1_ref, t1_ref, dinv_ref, u1_ref):
    deg = p0_ref[:, 0:1] + p1_ref[:, 0:1] + 1.0
    dinv = lax.rsqrt(jnp.broadcast_to(deg, (_BM, HID)))
    dinv_ref[...] = dinv
    u1_ref[...] = t1_ref[...] * dinv


def _k_mid(s0_ref, s1_ref, u_ref, dinv_ref, b_ref, w_ref, o_ref):
    dinv = dinv_ref[...]
    h = dinv * (s0_ref[...] + s1_ref[...] + u_ref[...]) + b_ref[...]
    h = jnp.maximum(h, 0.0)
    o_ref[...] = jnp.dot(h, w_ref[...],
                         preferred_element_type=jnp.float32) * dinv


def _k_emb(s0_ref, s1_ref, u_ref, dinv_ref, b_ref, o_ref):
    dinv = dinv_ref[...]
    emb = dinv * (s0_ref[...] + s1_ref[...] + u_ref[...]) + b_ref[...]
    o_ref[...] = emb * dinv


def _k_out(s0_ref, s1_ref, u_ref, dinv_ref, w_ref, b_ref, o_ref):
    ph = dinv_ref[...] * (s0_ref[...] + s1_ref[...] + u_ref[...])
    o_ref[...] = jnp.dot(ph, w_ref[...],
                         preferred_element_type=jnp.float32) + b_ref[...]


def kernel(x, edge_index, W1, b1, W2, b2, W3, b3, W4, b4):
    ei = edge_index.astype(jnp.int32)
    pad = EPAD - E
    pad_src = jnp.arange(pad, dtype=jnp.int32) % T
    srcp = jnp.concatenate([ei[0], pad_src]).reshape(ROWS, 128)
    pad_dst = N + (jnp.arange(pad, dtype=jnp.int32) % (NPAD - N))
    dstp = jnp.concatenate([ei[1], pad_dst]).reshape(ROWS, 128)
    zeros_h = jnp.zeros((RPT, HID), jnp.float32)
    zeros_8 = jnp.zeros((RPT, 8), jnp.float32)
    ones_8 = jnp.ones((128, 8), jnp.float32)

    losrc, lodst, hisrc, hidst, degp = _sc_partition(srcp, dstp, ones_8, zeros_8)
    losrc = losrc.reshape(NW, L1C, PCH, 128)
    lodst = lodst.reshape(NW, L1C, PCH, 128)
    hisrc = hisrc.reshape(NW, L2C, PCH, 128)
    hidst = hidst.reshape(NW, L2C, PCH, 128)

    t1 = _tc_call(_k_xw1, [_row_spec(IN), _full_spec(IN, HID)], HID)(x, W1)
    dinv, u1 = _tc_call(
        _k_dinv_u1, [_row_spec(8), _row_spec(8), _row_spec(HID)], HID, n_out=2,
    )(degp[0], degp[1], t1)

    sp = _sc_propagate(u1, losrc, lodst, hisrc, hidst, zeros_h)
    u2 = _tc_call(
        _k_mid,
        [_row_spec(HID)] * 4 + [_full_spec(1, HID), _full_spec(HID, HID)],
        HID,
    )(sp[0], sp[1], u1, dinv, b1.reshape(1, HID), W2)

    sp = _sc_propagate(u2, losrc, lodst, hisrc, hidst, zeros_h)
    u3 = _tc_call(
        _k_mid,
        [_row_spec(HID)] * 4 + [_full_spec(1, HID), _full_spec(HID, HID)],
        HID,
    )(sp[0], sp[1], u2, dinv, b2.reshape(1, HID), W3)

    sp = _sc_propagate(u3, losrc, lodst, hisrc, hidst, zeros_h)
    u4 = _tc_call(
        _k_emb, [_row_spec(HID)] * 4 + [_full_spec(1, HID)], HID,
    )(sp[0], sp[1], u3, dinv, b3.reshape(1, HID))

    sp = _sc_propagate(u4, losrc, lodst, hisrc, hidst, zeros_h)
    x_ = _tc_call(
        _k_out,
        [_row_spec(HID)] * 4 + [_full_spec(HID, IN), _full_spec(1, IN)],
        IN,
    )(sp[0], sp[1], u4, dinv, W4, b4.reshape(1, IN))
    return x_


# speedup vs baseline: 6.0954x; 1.0010x over previous
import functools

import jax
import jax.numpy as jnp
from jax import lax
from jax.experimental import pallas as pl
from jax.experimental.pallas import tpu as pltpu
from jax.experimental.pallas import tpu_sc as plsc

N = 10000
E = 320000
HID = 64
IN = 128

NC = 2
NS = 16
NW = NC * NS
CHUNK = 8
ROWS_PER_W = 80
ROWS = NW * ROWS_PER_W
EPAD = ROWS * 128
NPAD = 10112
RPT = NPAD // NS

T = 4096
L1 = 5120
L2 = 6656
PCH = 4
L1C = L1 // (PCH * 128)
L2C = L2 // (PCH * 128)
UPT = T // NS

_MESH = plsc.VectorSubcoreMesh(core_axis_name="c", subcore_axis_name="s")


@functools.partial(
    pl.kernel,
    out_type=[
        jax.ShapeDtypeStruct((NW, L1), jnp.int32),
        jax.ShapeDtypeStruct((NW, L1), jnp.int32),
        jax.ShapeDtypeStruct((NW, L2), jnp.int32),
        jax.ShapeDtypeStruct((NW, L2), jnp.int32),
        jax.ShapeDtypeStruct((NC, NPAD, 8), jnp.float32),
    ],
    mesh=_MESH,
    scratch_types=[
        pltpu.VMEM_SHARED((NPAD, 8), jnp.float32),
        pltpu.VMEM((CHUNK, 128), jnp.int32),
        pltpu.VMEM((CHUNK, 128), jnp.int32),
        pltpu.VMEM((128, 8), jnp.float32),
        pltpu.VMEM((L1 + 16,), jnp.int32),
        pltpu.VMEM((L1 + 16,), jnp.int32),
        pltpu.VMEM((L2 + 16,), jnp.int32),
        pltpu.VMEM((L2 + 16,), jnp.int32),
        pltpu.SemaphoreType.DMA,
    ],
    compiler_params=pltpu.CompilerParams(use_tc_tiling_on_sc=False,
                                         needs_layout_passes=False,
                                         skip_device_barrier=True),
)
def _sc_partition(src_hbm, dst_hbm, ones_hbm, zero_hbm,
                  losrc_hbm, lodst_hbm, hisrc_hbm, hidst_hbm, deg_hbm,
                  acc, sbuf, dbuf, obuf, ls, ld, hs, hd, dsem):
    c = lax.axis_index("c")
    s = lax.axis_index("s")
    w = c * NS + s
    pltpu.sync_copy(zero_hbm, acc.at[pl.ds(s * RPT, RPT)])
    pltpu.sync_copy(ones_hbm, obuf)
    plsc.subcore_barrier()
    row0 = w * ROWS_PER_W

    iota = lax.iota(jnp.int32, 16)
    rank = iota + 1

    def body(i, carry):
        off_lo, off_hi = carry
        base = row0 + i * CHUNK
        pltpu.sync_copy(src_hbm.at[pl.ds(base, CHUNK)], sbuf)
        pltpu.sync_copy(dst_hbm.at[pl.ds(base, CHUNK)], dbuf)
        degs = [
            pltpu.async_copy(obuf, acc.at[dbuf.at[j]], dsem, add=True)
            for j in range(CHUNK)
        ]
        for r in range(CHUNK):
            for g in range(8):
                sv = sbuf[r, pl.ds(g * 16, 16)]
                dv = dbuf[r, pl.ds(g * 16, 16)]
                m = sv < T
                cum = plsc.cumsum(m.astype(jnp.int32))
                n = plsc.all_reduce_population_count(m)
                pos_lo = jnp.where(m, off_lo + cum - 1, L1 + 15)
                plsc.store_scatter(ls, [pos_lo], sv)
                plsc.store_scatter(ld, [pos_lo], dv)
                pos_hi = jnp.where(m, L2 + 15, off_hi + (rank - cum) - 1)
                plsc.store_scatter(hs, [pos_hi], sv)
                plsc.store_scatter(hd, [pos_hi], dv)
                off_lo = jnp.minimum(off_lo + n, L1)
                off_hi = jnp.minimum(off_hi + (16 - n), L2)
        for cp in degs:
            cp.wait()
        return off_lo, off_hi

    zoff = jnp.zeros((16,), jnp.int32)
    off_lo, off_hi = lax.fori_loop(0, ROWS_PER_W // CHUNK, body, (zoff, zoff))

    def padlo(i, off):
        pos = jnp.minimum(off + i * 16 + iota, L1 + 15)
        sp_ = (i * 16 + iota) % T
        dp_ = N + ((i * 16 + iota) % (NPAD - N))
        plsc.store_scatter(ls, [pos], sp_)
        plsc.store_scatter(ld, [pos], dp_)
        return off

    lax.fori_loop(0, L1 // 16 + 1, padlo, off_lo)

    def padhi(i, off):
        pos = jnp.minimum(off + i * 16 + iota, L2 + 15)
        sp_ = T + ((i * 16 + iota) % (N - T))
        dp_ = N + ((i * 16 + iota) % (NPAD - N))
        plsc.store_scatter(hs, [pos], sp_)
        plsc.store_scatter(hd, [pos], dp_)
        return off

    lax.fori_loop(0, L2 // 16 + 1, padhi, off_hi)

    pltpu.sync_copy(ls.at[pl.ds(0, L1)], losrc_hbm.at[w])
    pltpu.sync_copy(ld.at[pl.ds(0, L1)], lodst_hbm.at[w])
    pltpu.sync_copy(hs.at[pl.ds(0, L2)], hisrc_hbm.at[w])
    pltpu.sync_copy(hd.at[pl.ds(0, L2)], hidst_hbm.at[w])
    plsc.subcore_barrier()
    pltpu.sync_copy(acc.at[pl.ds(s * RPT, RPT)],
                    deg_hbm.at[c, pl.ds(s * RPT, RPT)])


@functools.partial(
    pl.kernel,
    out_type=jax.ShapeDtypeStruct((NC, NPAD, HID), jnp.float32),
    mesh=_MESH,
    scratch_types=[
        pltpu.VMEM_SHARED((NPAD, HID), jnp.float32),
        pltpu.VMEM_SHARED((T, HID), jnp.float32),
        pltpu.VMEM((PCH, 128), jnp.int32),
        pltpu.VMEM((2, PCH, 128), jnp.int32),
        pltpu.VMEM((2, PCH, 128, HID), jnp.float32),
        pltpu.SemaphoreType.DMA,
        pltpu.SemaphoreType.DMA,
    ],
    compiler_params=pltpu.CompilerParams(use_tc_tiling_on_sc=False,
                                         skip_device_barrier=True),
)
def _sc_propagate(u_hbm, losrc_hbm, lodst_hbm, hisrc_hbm, hidst_hbm, zero_hbm,
                  out_hbm, acc, u_spm, sbuf, dbuf, gbuf, gsem, ssem):
    c = lax.axis_index("c")
    s = lax.axis_index("s")
    pltpu.sync_copy(zero_hbm, acc.at[pl.ds(s * RPT, RPT)])

    @pl.when(c == 1)
    def _():
        pltpu.sync_copy(u_hbm.at[pl.ds(s * UPT, UPT)],
                        u_spm.at[pl.ds(s * UPT, UPT)])

    plsc.subcore_barrier()

    def run(src_lists, dst_lists, n_chunks, table):
        pend = [None, None]
        seq = [(2 * s + k, i) for k in range(2) for i in range(n_chunks)]
        for t, (w2, i) in enumerate(seq):
            p = t % 2
            if pend[p] is not None:
                for cp in pend[p]:
                    cp.wait()
            pltpu.sync_copy(src_lists.at[w2, i], sbuf)
            pltpu.sync_copy(dst_lists.at[w2, i], dbuf.at[p])
            gs = [
                pltpu.async_copy(table.at[sbuf.at[j]], gbuf.at[p, j], gsem)
                for j in range(PCH)
            ]
            for cp in gs:
                cp.wait()
            pend[p] = [
                pltpu.async_copy(gbuf.at[p, j], acc.at[dbuf.at[p, j]],
                                 ssem, add=True)
                for j in range(PCH)
            ]
        for ss in pend:
            if ss is not None:
                for cp in ss:
                    cp.wait()

    @pl.when(c == 1)
    def _():
        run(losrc_hbm, lodst_hbm, L1C, u_spm)

    @pl.when(c == 0)
    def _():
        run(hisrc_hbm, hidst_hbm, L2C, u_hbm)

    plsc.subcore_barrier()
    pltpu.sync_copy(acc.at[pl.ds(s * RPT, RPT)],
                    out_hbm.at[c, pl.ds(s * RPT, RPT)])


_BM = 1000


def _row_spec(d):
    return pl.BlockSpec((_BM, d), lambda i: (i, 0))


def _full_spec(r, d):
    return pl.BlockSpec((r, d), lambda i: (0, 0))


def _tc_call(body, in_specs, out_dim, n_out=1):
    if n_out == 1:
        out_shape = jax.ShapeDtypeStruct((N, out_dim), jnp.float32)
        out_specs = _row_spec(out_dim)
    else:
        out_shape = [jax.ShapeDtypeStruct((N, out_dim), jnp.float32)] * n_out
        out_specs = [_row_spec(out_dim)] * n_out
    return pl.pallas_call(
        body,
        grid=(N // _BM,),
        in_specs=in_specs,
        out_specs=out_specs,
        out_shape=out_shape,
    )


def _k_xw1(x_ref, w_ref, o_ref):
    o_ref[...] = jnp.dot(x_ref[...], w_ref[...],
                         preferred_element_type=jnp.float32)


def _k_dinv_u1(p0_ref, p1_ref, t1_ref, dinv_ref, u1_ref):
    deg = p0_ref[:, 0:1] + p1_ref[:, 0:1] + 1.0
    dinv = lax.rsqrt(jnp.broadcast_to(deg, (_BM, HID)))
    dinv_ref[...] = dinv
    u1_ref[...] = t1_ref[...] * dinv


def _k_mid(s0_ref, s1_ref, u_ref, dinv_ref, b_ref, w_ref, o_ref):
    dinv = dinv_ref[...]
    h = dinv * (s0_ref[...] + s1_ref[...] + u_ref[...]) + b_ref[...]
    h = jnp.maximum(h, 0.0)
    o_ref[...] = jnp.dot(h, w_ref[...],
                         preferred_element_type=jnp.float32) * dinv


def _k_emb(s0_ref, s1_ref, u_ref, dinv_ref, b_ref, o_ref):
    dinv = dinv_ref[...]
    emb = dinv * (s0_ref[...] + s1_ref[...] + u_ref[...]) + b_ref[...]
    o_ref[...] = emb * dinv


def _k_out(s0_ref, s1_ref, u_ref, dinv_ref, w_ref, b_ref, o_ref):
    ph = dinv_ref[...] * (s0_ref[...] + s1_ref[...] + u_ref[...])
    o_ref[...] = jnp.dot(ph, w_ref[...],
                         preferred_element_type=jnp.float32) + b_ref[...]


def kernel(x, edge_index, W1, b1, W2, b2, W3, b3, W4, b4):
    ei = edge_index.astype(jnp.int32)
    pad = EPAD - E
    pad_src = jnp.arange(pad, dtype=jnp.int32) % T
    srcp = jnp.concatenate([ei[0], pad_src]).reshape(ROWS, 128)
    pad_dst = N + (jnp.arange(pad, dtype=jnp.int32) % (NPAD - N))
    dstp = jnp.concatenate([ei[1], pad_dst]).reshape(ROWS, 128)
    zeros_h = jnp.zeros((RPT, HID), jnp.float32)
    zeros_8 = jnp.zeros((RPT, 8), jnp.float32)
    ones_8 = jnp.ones((128, 8), jnp.float32)

    losrc, lodst, hisrc, hidst, degp = _sc_partition(srcp, dstp, ones_8, zeros_8)
    losrc = losrc.reshape(NW, L1C, PCH, 128)
    lodst = lodst.reshape(NW, L1C, PCH, 128)
    hisrc = hisrc.reshape(NW, L2C, PCH, 128)
    hidst = hidst.reshape(NW, L2C, PCH, 128)

    t1 = _tc_call(_k_xw1, [_row_spec(IN), _full_spec(IN, HID)], HID)(x, W1)
    dinv, u1 = _tc_call(
        _k_dinv_u1, [_row_spec(8), _row_spec(8), _row_spec(HID)], HID, n_out=2,
    )(degp[0], degp[1], t1)

    sp = _sc_propagate(u1, losrc, lodst, hisrc, hidst, zeros_h)
    u2 = _tc_call(
        _k_mid,
        [_row_spec(HID)] * 4 + [_full_spec(1, HID), _full_spec(HID, HID)],
        HID,
    )(sp[0], sp[1], u1, dinv, b1.reshape(1, HID), W2)

    sp = _sc_propagate(u2, losrc, lodst, hisrc, hidst, zeros_h)
    u3 = _tc_call(
        _k_mid,
        [_row_spec(HID)] * 4 + [_full_spec(1, HID), _full_spec(HID, HID)],
        HID,
    )(sp[0], sp[1], u2, dinv, b2.reshape(1, HID), W3)

    sp = _sc_propagate(u3, losrc, lodst, hisrc, hidst, zeros_h)
    u4 = _tc_call(
        _k_emb, [_row_spec(HID)] * 4 + [_full_spec(1, HID)], HID,
    )(sp[0], sp[1], u3, dinv, b3.reshape(1, HID))

    sp = _sc_propagate(u4, losrc, lodst, hisrc, hidst, zeros_h)
    x_ = _tc_call(
        _k_out,
        [_row_spec(HID)] * 4 + [_full_spec(HID, IN), _full_spec(1, IN)],
        IN,
    )(sp[0], sp[1], u4, dinv, W4, b4.reshape(1, IN))
    return x_


# speedup vs baseline: 6.4730x; 1.0619x over previous
import functools

import jax
import jax.numpy as jnp
from jax import lax
from jax.experimental import pallas as pl
from jax.experimental.pallas import tpu as pltpu
from jax.experimental.pallas import tpu_sc as plsc

N = 10000
E = 320000
HID = 64
IN = 128

NC = 2
NS = 16
NW = NC * NS
CHUNK = 8
ROWS_PER_W = 80
ROWS = NW * ROWS_PER_W
EPAD = ROWS * 128
NPAD = 10112
RPT = NPAD // NS

T = 4096
L1 = 5120
L2 = 6656
PCH = 4
L1C = L1 // (PCH * 128)
L2C = L2 // (PCH * 128)
UPT = T // NS

_MESH = plsc.VectorSubcoreMesh(core_axis_name="c", subcore_axis_name="s")


@functools.partial(
    pl.kernel,
    out_type=[
        jax.ShapeDtypeStruct((NW, L1), jnp.int32),
        jax.ShapeDtypeStruct((NW, L1), jnp.int32),
        jax.ShapeDtypeStruct((NW, L2), jnp.int32),
        jax.ShapeDtypeStruct((NW, L2), jnp.int32),
        jax.ShapeDtypeStruct((NC, NPAD, 8), jnp.float32),
    ],
    mesh=_MESH,
    scratch_types=[
        pltpu.VMEM_SHARED((NPAD, 8), jnp.float32),
        pltpu.VMEM((CHUNK, 128), jnp.int32),
        pltpu.VMEM((CHUNK, 128), jnp.int32),
        pltpu.VMEM((128, 8), jnp.float32),
        pltpu.VMEM((L1 + 16,), jnp.int32),
        pltpu.VMEM((L1 + 16,), jnp.int32),
        pltpu.VMEM((L2 + 16,), jnp.int32),
        pltpu.VMEM((L2 + 16,), jnp.int32),
        pltpu.SemaphoreType.DMA,
    ],
    compiler_params=pltpu.CompilerParams(use_tc_tiling_on_sc=False,
                                         needs_layout_passes=False),
)
def _sc_partition(src_hbm, dst_hbm, ones_hbm, zero_hbm,
                  losrc_hbm, lodst_hbm, hisrc_hbm, hidst_hbm, deg_hbm,
                  acc, sbuf, dbuf, obuf, ls, ld, hs, hd, dsem):
    c = lax.axis_index("c")
    s = lax.axis_index("s")
    w = c * NS + s
    pltpu.sync_copy(zero_hbm, acc.at[pl.ds(s * RPT, RPT)])
    pltpu.sync_copy(ones_hbm, obuf)
    plsc.subcore_barrier()
    row0 = w * ROWS_PER_W

    iota = lax.iota(jnp.int32, 16)
    rank = iota + 1

    def body(i, carry):
        off_lo, off_hi = carry
        base = row0 + i * CHUNK
        pltpu.sync_copy(src_hbm.at[pl.ds(base, CHUNK)], sbuf)
        pltpu.sync_copy(dst_hbm.at[pl.ds(base, CHUNK)], dbuf)
        degs = [
            pltpu.async_copy(obuf, acc.at[dbuf.at[j]], dsem, add=True)
            for j in range(CHUNK)
        ]
        for r in range(CHUNK):
            for g in range(8):
                sv = sbuf[r, pl.ds(g * 16, 16)]
                dv = dbuf[r, pl.ds(g * 16, 16)]
                m = sv < T
                cum = plsc.cumsum(m.astype(jnp.int32))
                n = plsc.all_reduce_population_count(m)
                pos_lo = jnp.where(m, off_lo + cum - 1, L1 + 15)
                plsc.store_scatter(ls, [pos_lo], sv)
                plsc.store_scatter(ld, [pos_lo], dv)
                pos_hi = jnp.where(m, L2 + 15, off_hi + (rank - cum) - 1)
                plsc.store_scatter(hs, [pos_hi], sv)
                plsc.store_scatter(hd, [pos_hi], dv)
                off_lo = jnp.minimum(off_lo + n, L1)
                off_hi = jnp.minimum(off_hi + (16 - n), L2)
        for cp in degs:
            cp.wait()
        return off_lo, off_hi

    zoff = jnp.zeros((16,), jnp.int32)
    off_lo, off_hi = lax.fori_loop(0, ROWS_PER_W // CHUNK, body, (zoff, zoff))

    def padlo(i, off):
        pos = jnp.minimum(off + i * 16 + iota, L1 + 15)
        sp_ = (i * 16 + iota) % T
        dp_ = N + ((i * 16 + iota) % (NPAD - N))
        plsc.store_scatter(ls, [pos], sp_)
        plsc.store_scatter(ld, [pos], dp_)
        return off

    lax.fori_loop(0, L1 // 16 + 1, padlo, off_lo)

    def padhi(i, off):
        pos = jnp.minimum(off + i * 16 + iota, L2 + 15)
        sp_ = T + ((i * 16 + iota) % (N - T))
        dp_ = N + ((i * 16 + iota) % (NPAD - N))
        plsc.store_scatter(hs, [pos], sp_)
        plsc.store_scatter(hd, [pos], dp_)
        return off

    lax.fori_loop(0, L2 // 16 + 1, padhi, off_hi)

    pltpu.sync_copy(ls.at[pl.ds(0, L1)], losrc_hbm.at[w])
    pltpu.sync_copy(ld.at[pl.ds(0, L1)], lodst_hbm.at[w])
    pltpu.sync_copy(hs.at[pl.ds(0, L2)], hisrc_hbm.at[w])
    pltpu.sync_copy(hd.at[pl.ds(0, L2)], hidst_hbm.at[w])
    plsc.subcore_barrier()
    pltpu.sync_copy(acc.at[pl.ds(s * RPT, RPT)],
                    deg_hbm.at[c, pl.ds(s * RPT, RPT)])


@functools.partial(
    pl.kernel,
    out_type=jax.ShapeDtypeStruct((NC, NPAD, HID), jnp.float32),
    mesh=_MESH,
    scratch_types=[
        pltpu.VMEM_SHARED((NPAD, HID), jnp.float32),
        pltpu.VMEM_SHARED((T, HID), jnp.float32),
        pltpu.VMEM((PCH, 128), jnp.int32),
        pltpu.VMEM((2, PCH, 128), jnp.int32),
        pltpu.VMEM((2, PCH, 128, HID), jnp.float32),
        pltpu.SemaphoreType.DMA,
        pltpu.SemaphoreType.DMA,
    ],
    compiler_params=pltpu.CompilerParams(use_tc_tiling_on_sc=False),
)
def _sc_propagate(u_hbm, losrc_hbm, lodst_hbm, hisrc_hbm, hidst_hbm, zero_hbm,
                  out_hbm, acc, u_spm, sbuf, dbuf, gbuf, gsem, ssem):
    c = lax.axis_index("c")
    s = lax.axis_index("s")
    pltpu.sync_copy(zero_hbm, acc.at[pl.ds(s * RPT, RPT)])

    @pl.when(c == 1)
    def _():
        pltpu.sync_copy(u_hbm.at[pl.ds(s * UPT, UPT)],
                        u_spm.at[pl.ds(s * UPT, UPT)])

    plsc.subcore_barrier()

    def run(src_lists, dst_lists, n_chunks, table):
        pend = [None, None]
        seq = [(2 * s + k, i) for k in range(2) for i in range(n_chunks)]
        for t, (w2, i) in enumerate(seq):
            p = t % 2
            if pend[p] is not None:
                for cp in pend[p]:
                    cp.wait()
            pltpu.sync_copy(src_lists.at[w2, i], sbuf)
            pltpu.sync_copy(dst_lists.at[w2, i], dbuf.at[p])
            gs = [
                pltpu.async_copy(table.at[sbuf.at[j]], gbuf.at[p, j], gsem)
                for j in range(PCH)
            ]
            for cp in gs:
                cp.wait()
            pend[p] = [
                pltpu.async_copy(gbuf.at[p, j], acc.at[dbuf.at[p, j]],
                                 ssem, add=True)
                for j in range(PCH)
            ]
        for ss in pend:
            if ss is not None:
                for cp in ss:
                    cp.wait()

    @pl.when(c == 1)
    def _():
        run(losrc_hbm, lodst_hbm, L1C, u_spm)

    @pl.when(c == 0)
    def _():
        run(hisrc_hbm, hidst_hbm, L2C, u_hbm)

    plsc.subcore_barrier()
    pltpu.sync_copy(acc.at[pl.ds(s * RPT, RPT)],
                    out_hbm.at[c, pl.ds(s * RPT, RPT)])


_BM = 1000


def _row_spec(d):
    return pl.BlockSpec((_BM, d), lambda i: (i, 0))


def _part_spec(p, d):
    return pl.BlockSpec((1, _BM, d), lambda i, _p=p: (_p, i, 0))


def _full_spec(r, d):
    return pl.BlockSpec((r, d), lambda i: (0, 0))


def _tc_call(body, in_specs, out_dim, n_out=1):
    if n_out == 1:
        out_shape = jax.ShapeDtypeStruct((N, out_dim), jnp.float32)
        out_specs = _row_spec(out_dim)
    else:
        out_shape = [jax.ShapeDtypeStruct((N, out_dim), jnp.float32)] * n_out
        out_specs = [_row_spec(out_dim)] * n_out
    return pl.pallas_call(
        body,
        grid=(N // _BM,),
        in_specs=in_specs,
        out_specs=out_specs,
        out_shape=out_shape,
    )


def _k_xw1(x_ref, w_ref, o_ref):
    o_ref[...] = jnp.dot(x_ref[...], w_ref[...],
                         preferred_element_type=jnp.float32)


def _k_dinv_u1(p0_ref, p1_ref, t1_ref, dinv_ref, u1_ref):
    deg = p0_ref[0, :, 0:1] + p1_ref[0, :, 0:1] + 1.0
    dinv = lax.rsqrt(jnp.broadcast_to(deg, (_BM, HID)))
    dinv_ref[...] = dinv
    u1_ref[...] = t1_ref[...] * dinv


def _k_mid(s0_ref, s1_ref, u_ref, dinv_ref, b_ref, w_ref, o_ref):
    dinv = dinv_ref[...]
    h = dinv * (s0_ref[0] + s1_ref[0] + u_ref[...]) + b_ref[...]
    h = jnp.maximum(h, 0.0)
    o_ref[...] = jnp.dot(h, w_ref[...],
                         preferred_element_type=jnp.float32) * dinv


def _k_emb(s0_ref, s1_ref, u_ref, dinv_ref, b_ref, o_ref):
    dinv = dinv_ref[...]
    emb = dinv * (s0_ref[0] + s1_ref[0] + u_ref[...]) + b_ref[...]
    o_ref[...] = emb * dinv


def _k_out(s0_ref, s1_ref, u_ref, dinv_ref, w_ref, b_ref, o_ref):
    ph = dinv_ref[...] * (s0_ref[0] + s1_ref[0] + u_ref[...])
    o_ref[...] = jnp.dot(ph, w_ref[...],
                         preferred_element_type=jnp.float32) + b_ref[...]


def kernel(x, edge_index, W1, b1, W2, b2, W3, b3, W4, b4):
    ei = edge_index.astype(jnp.int32)
    pad = EPAD - E
    pad_src = jnp.arange(pad, dtype=jnp.int32) % T
    srcp = jnp.concatenate([ei[0], pad_src]).reshape(ROWS, 128)
    pad_dst = N + (jnp.arange(pad, dtype=jnp.int32) % (NPAD - N))
    dstp = jnp.concatenate([ei[1], pad_dst]).reshape(ROWS, 128)
    zeros_h = jnp.zeros((RPT, HID), jnp.float32)
    zeros_8 = jnp.zeros((RPT, 8), jnp.float32)
    ones_8 = jnp.ones((128, 8), jnp.float32)

    losrc, lodst, hisrc, hidst, degp = _sc_partition(srcp, dstp, ones_8, zeros_8)
    losrc = losrc.reshape(NW, L1C, PCH, 128)
    lodst = lodst.reshape(NW, L1C, PCH, 128)
    hisrc = hisrc.reshape(NW, L2C, PCH, 128)
    hidst = hidst.reshape(NW, L2C, PCH, 128)

    t1 = _tc_call(_k_xw1, [_row_spec(IN), _full_spec(IN, HID)], HID)(x, W1)
    dinv, u1 = _tc_call(
        _k_dinv_u1,
        [_part_spec(0, 8), _part_spec(1, 8), _row_spec(HID)], HID, n_out=2,
    )(degp, degp, t1)

    mid_specs = ([_part_spec(0, HID), _part_spec(1, HID), _row_spec(HID),
                  _row_spec(HID), _full_spec(1, HID), _full_spec(HID, HID)])

    sp = _sc_propagate(u1, losrc, lodst, hisrc, hidst, zeros_h)
    u2 = _tc_call(_k_mid, mid_specs, HID)(
        sp, sp, u1, dinv, b1.reshape(1, HID), W2)

    sp = _sc_propagate(u2, losrc, lodst, hisrc, hidst, zeros_h)
    u3 = _tc_call(_k_mid, mid_specs, HID)(
        sp, sp, u2, dinv, b2.reshape(1, HID), W3)

    sp = _sc_propagate(u3, losrc, lodst, hisrc, hidst, zeros_h)
    u4 = _tc_call(_k_emb, mid_specs[:5], HID)(
        sp, sp, u3, dinv, b3.reshape(1, HID))

    sp = _sc_propagate(u4, losrc, lodst, hisrc, hidst, zeros_h)
    x_ = _tc_call(
        _k_out,
        mid_specs[:4] + [_full_spec(HID, IN), _full_spec(1, IN)],
        IN,
    )(sp, sp, u4, dinv, W4, b4.reshape(1, IN))
    return x_
